# baseline (speedup 1.0000x reference)
import functools

import jax
import jax.numpy as jnp
from jax import lax
from jax.experimental import pallas as pl
from jax.experimental.pallas import tpu as pltpu
from jax.experimental.pallas import tpu_sc as plsc

D = 64
CHUNK = 640
NBUF = 2
NC = 2
NS = 16
NW = NC * NS


def kernel(x, pe):
    B = x.size
    assert B % (NW * NBUF * CHUNK) == 0
    n_bodies = B // (NW * NBUF * CHUNK)
    n_rows, seq = x.shape
    half = seq // 2
    xf = x.reshape(n_rows, 2, half).transpose(0, 2, 1).reshape(B)
    per_w = B // NW

    mesh = plsc.VectorSubcoreMesh(
        core_axis_name="c", subcore_axis_name="s", num_cores=NC, num_subcores=NS
    )

    @functools.partial(
        pl.kernel,
        mesh=mesh,
        compiler_params=pltpu.CompilerParams(use_tc_tiling_on_sc=False),
        out_type=jax.ShapeDtypeStruct((B, D), jnp.float32),
        scratch_types=[
            pltpu.VMEM((NBUF, CHUNK), jnp.int32),
            pltpu.VMEM((NBUF, CHUNK, D), jnp.float32),
            pltpu.SemaphoreType.DMA,
            pltpu.SemaphoreType.DMA,
            pltpu.SemaphoreType.DMA,
            pltpu.SemaphoreType.DMA,
        ],
    )
    def gather_kernel(idx_hbm, table_hbm, out_hbm, idx_v, rows_v,
                      gsem0, gsem1, osem0, osem1):
        wid = lax.axis_index("s") * NC + lax.axis_index("c")
        base = wid * per_w
        gsems = (gsem0, gsem1)
        osems = (osem0, osem1)

        def fire_gather(r0, b):
            pltpu.sync_copy(idx_hbm.at[pl.ds(r0, CHUNK)], idx_v.at[b])
            return pltpu.async_copy(
                table_hbm.at[idx_v.at[b]], rows_v.at[b], gsems[b]
            )

        def drain_out(b):
            pltpu.make_async_copy(
                rows_v.at[b], out_hbm.at[pl.ds(0, CHUNK)], osems[b]
            ).wait()

        def body(g, carry):
            r0 = base + g * (NBUF * CHUNK)
            r1 = r0 + CHUNK

            @pl.when(g > 0)
            def _():
                drain_out(0)

            d0 = fire_gather(r0, 0)

            @pl.when(g > 0)
            def _():
                drain_out(1)

            d1 = fire_gather(r1, 1)
            d0.wait()
            pltpu.async_copy(rows_v.at[0], out_hbm.at[pl.ds(r0, CHUNK)], osems[0])
            d1.wait()
            pltpu.async_copy(rows_v.at[1], out_hbm.at[pl.ds(r1, CHUNK)], osems[1])
            return carry

        lax.fori_loop(0, n_bodies, body, 0)
        drain_out(0)
        drain_out(1)

    out = gather_kernel(xf, pe)

    BI = 64
    v = out.reshape(n_rows, half, 2 * D)

    def _compact(v_ref, o_ref):
        v = v_ref[...]
        o_ref[:, :half, :] = v[:, :, :D]
        o_ref[:, half:, :] = v[:, :, D:]

    final = pl.pallas_call(
        _compact,
        grid=(n_rows // BI,),
        in_specs=[pl.BlockSpec((BI, half, 2 * D), lambda i: (i, 0, 0))],
        out_specs=pl.BlockSpec((BI, seq, D), lambda i: (i, 0, 0)),
        out_shape=jax.ShapeDtypeStruct((n_rows, seq, D), jnp.float32),
    )(v)
    return final


# speedup vs baseline: 1.6622x; 1.6622x over previous
import functools

import jax
import jax.numpy as jnp
import numpy as np
from jax import lax
from jax.experimental import pallas as pl
from jax.experimental.pallas import tpu as pltpu
from jax.experimental.pallas import tpu_sc as plsc

D = 64
SEQ = 200
NI = 4
CHUNK = NI * SEQ
NVEC = CHUNK // 16
NBUF = 2
NC = 2
NS = 16
NW = NC * NS


def kernel(x, pe):
    B = x.size
    n_rows, seq = x.shape
    assert seq == SEQ and B % (NW * NBUF * CHUNK) == 0
    n_bodies = B // (NW * NBUF * CHUNK)
    half = seq // 2
    xf = x.reshape(B)
    per_w = B // NW
    _q = np.arange(CHUNK, dtype=np.int32)
    _k = _q % SEQ
    perm = jnp.asarray((_q // SEQ) * SEQ + (_k % 2) * half + _k // 2)

    mesh = plsc.VectorSubcoreMesh(
        core_axis_name="c", subcore_axis_name="s", num_cores=NC, num_subcores=NS
    )

    @functools.partial(
        pl.kernel,
        mesh=mesh,
        compiler_params=pltpu.CompilerParams(
            use_tc_tiling_on_sc=False, needs_layout_passes=False
        ),
        out_type=jax.ShapeDtypeStruct((B, D), jnp.float32),
        scratch_types=[
            pltpu.VMEM((CHUNK,), jnp.int32),
            pltpu.VMEM((CHUNK,), jnp.int32),
            pltpu.VMEM((CHUNK,), jnp.int32),
            pltpu.VMEM((CHUNK,), jnp.int32),
            pltpu.VMEM((NBUF, CHUNK, D), jnp.float32),
            pltpu.SemaphoreType.DMA,
            pltpu.SemaphoreType.DMA,
            pltpu.SemaphoreType.DMA,
            pltpu.SemaphoreType.DMA,
        ],
    )
    def gather_kernel(idx_hbm, table_hbm, perm_hbm, out_hbm, srcs, idx_raw,
                      idx_v0, idx_v1, rows_v, gsem0, gsem1, osem0, osem1):
        wid = lax.axis_index("s") * NC + lax.axis_index("c")
        base = wid * per_w
        idx_vs = (idx_v0, idx_v1)
        gsems = (gsem0, gsem1)
        osems = (osem0, osem1)

        pltpu.sync_copy(perm_hbm, srcs)

        def fire_gather(r0, b):
            pltpu.sync_copy(idx_hbm.at[pl.ds(r0, CHUNK)], idx_raw)
            for v in range(NVEC):
                sv = srcs[pl.ds(16 * v, 16)]
                idx_vs[b][pl.ds(16 * v, 16)] = plsc.load_gather(idx_raw, [sv])
            return pltpu.async_copy(
                table_hbm.at[idx_vs[b]], rows_v.at[b], gsems[b]
            )

        def drain_out(b):
            pltpu.make_async_copy(
                rows_v.at[b], out_hbm.at[pl.ds(0, CHUNK)], osems[b]
            ).wait()

        def body(g, carry):
            r0 = base + g * (NBUF * CHUNK)
            r1 = r0 + CHUNK

            @pl.when(g > 0)
            def _():
                drain_out(0)

            d0 = fire_gather(r0, 0)

            @pl.when(g > 0)
            def _():
                drain_out(1)

            d1 = fire_gather(r1, 1)
            d0.wait()
            pltpu.async_copy(rows_v.at[0], out_hbm.at[pl.ds(r0, CHUNK)], osems[0])
            d1.wait()
            pltpu.async_copy(rows_v.at[1], out_hbm.at[pl.ds(r1, CHUNK)], osems[1])
            return carry

        lax.fori_loop(0, n_bodies, body, 0)
        drain_out(0)
        drain_out(1)

    out = gather_kernel(xf, pe, perm)

    BI = 64
    v = out.reshape(n_rows, half, 2 * D)

    def _compact(v_ref, o_ref):
        vv = v_ref[...]
        o_ref[:, :half, :] = vv[:, :, :D]
        o_ref[:, half:, :] = vv[:, :, D:]

    final = pl.pallas_call(
        _compact,
        grid=(n_rows // BI,),
        in_specs=[pl.BlockSpec((BI, half, 2 * D), lambda i: (i, 0, 0))],
        out_specs=pl.BlockSpec((BI, seq, D), lambda i: (i, 0, 0)),
        out_shape=jax.ShapeDtypeStruct((n_rows, seq, D), jnp.float32),
    )(v)
    return final


# speedup vs baseline: 2.1478x; 1.2922x over previous
import functools

import jax
import jax.numpy as jnp
from jax import lax
from jax.experimental import pallas as pl
from jax.experimental.pallas import tpu as pltpu
from jax.experimental.pallas import tpu_sc as plsc

D = 64
CHUNK = 800
NBUF = 2
NC = 2
NS = 16
NW = NC * NS


def kernel(x, pe):
    B = x.size
    assert B % (NW * NBUF * CHUNK) == 0
    n_bodies = B // (NW * NBUF * CHUNK)
    xf = x.reshape(B)
    per_w = B // NW

    mesh = plsc.VectorSubcoreMesh(
        core_axis_name="c", subcore_axis_name="s", num_cores=NC, num_subcores=NS
    )

    @functools.partial(
        pl.kernel,
        mesh=mesh,
        compiler_params=pltpu.CompilerParams(use_tc_tiling_on_sc=False),
        out_type=jax.ShapeDtypeStruct((B, D), jnp.float32),
        scratch_types=[
            pltpu.VMEM((NBUF, CHUNK), jnp.int32),
            pltpu.VMEM((NBUF, CHUNK, D), jnp.float32),
            pltpu.SemaphoreType.DMA,
            pltpu.SemaphoreType.DMA,
            pltpu.SemaphoreType.DMA,
            pltpu.SemaphoreType.DMA,
        ],
    )
    def gather_kernel(idx_hbm, table_hbm, out_hbm, idx_v, rows_v,
                      gsem0, gsem1, osem0, osem1):
        wid = lax.axis_index("s") * NC + lax.axis_index("c")
        base = wid * per_w
        gsems = (gsem0, gsem1)
        osems = (osem0, osem1)

        def fire_gather(r0, b):
            pltpu.sync_copy(idx_hbm.at[pl.ds(r0, CHUNK)], idx_v.at[b])
            return pltpu.async_copy(
                table_hbm.at[idx_v.at[b]], rows_v.at[b], gsems[b]
            )

        def drain_out(b):
            pltpu.make_async_copy(
                rows_v.at[b], out_hbm.at[pl.ds(0, CHUNK)], osems[b]
            ).wait()

        def body(g, carry):
            r0 = base + g * (NBUF * CHUNK)
            r1 = r0 + CHUNK

            @pl.when(g > 0)
            def _():
                drain_out(0)

            d0 = fire_gather(r0, 0)

            @pl.when(g > 0)
            def _():
                drain_out(1)

            d1 = fire_gather(r1, 1)
            d0.wait()
            pltpu.async_copy(rows_v.at[0], out_hbm.at[pl.ds(r0, CHUNK)], osems[0])
            d1.wait()
            pltpu.async_copy(rows_v.at[1], out_hbm.at[pl.ds(r1, CHUNK)], osems[1])
            return carry

        lax.fori_loop(0, n_bodies, body, 0)
        drain_out(0)
        drain_out(1)

    out = gather_kernel(xf, pe)
    return out.reshape(x.shape + (D,))
